# baseline (device time: 23582 ns/iter reference)
import jax
import jax.numpy as jnp
from jax import lax
from jax.experimental import pallas as pl
from jax.experimental.pallas import tpu as pltpu

N_DEV = 4
B, Sq, SKV_PER, HQ_LOC, DH = 2, 128, 128, 4, 64
HQ = 16
D_MODEL = 512
SKV = SKV_PER * N_DEV
HD_LOC = HQ_LOC * DH
HP = 2 * DH
KVROWS = 2 * SKV_PER
SCALE = 0.125


def kernel(x, Wq, K_ext, V_ext, Wo):
    K2 = K_ext.reshape(B * SKV_PER, HQ * DH)
    V2 = V_ext.reshape(B * SKV_PER, HQ * DH)

    def body(x_ref, wq_ref, k_ref, v_ref, wo_ref, out_ref,
             kv16, kv_all, my_part, acc,
             kv_send, kv_recv, loc_sem, ar_send, ar_recv):
        my = lax.axis_index("i")

        barrier_sem = pltpu.get_barrier_semaphore()
        for d in range(1, N_DEV):
            peer = lax.rem(my + d, N_DEV)
            pl.semaphore_signal(
                barrier_sem, inc=1,
                device_id=(peer,), device_id_type=pl.DeviceIdType.MESH,
            )

        kv16[0:SKV_PER] = k_ref[0:SKV_PER].astype(jnp.bfloat16)
        kv16[SKV_PER:KVROWS] = v_ref[0:SKV_PER].astype(jnp.bfloat16)

        pl.semaphore_wait(barrier_sem, N_DEV - 1)

        kv_rdmas = {(b, p): [] for b in range(B) for p in range(2)}

        def fire_kv(b):
            for p in range(2):
                for d in range(1, N_DEV):
                    peer = lax.rem(my + d, N_DEV)
                    slot = N_DEV - d
                    r = pltpu.make_async_remote_copy(
                        src_ref=kv16.at[pl.ds(b * KVROWS, KVROWS),
                                        pl.ds(peer * HD_LOC + p * HP, HP)],
                        dst_ref=kv_all.at[slot, pl.ds(b * KVROWS, KVROWS),
                                          pl.ds(p * HP, HP)],
                        send_sem=kv_send.at[d - 1, b, p],
                        recv_sem=kv_recv.at[slot - 1, b, p],
                        device_id=(peer,), device_id_type=pl.DeviceIdType.MESH,
                    )
                    r.start()
                    kv_rdmas[(b, p)].append(r)

        fire_kv(0)

        kv16[KVROWS:KVROWS + SKV_PER] = k_ref[SKV_PER:2 * SKV_PER].astype(
            jnp.bfloat16)
        kv16[KVROWS + SKV_PER:2 * KVROWS] = v_ref[SKV_PER:2 * SKV_PER].astype(
            jnp.bfloat16)
        fire_kv(1)

        own_kv = pltpu.make_async_copy(
            kv16.at[:, pl.ds(my * HD_LOC, HD_LOC)], kv_all.at[0], loc_sem)
        own_kv.start()

        xv = x_ref[...].reshape(B * Sq, D_MODEL)
        q = jnp.dot(xv, wq_ref[...],
                    preferred_element_type=jnp.float32).astype(jnp.bfloat16)

        qi = lax.broadcasted_iota(jnp.int32, (Sq, SKV), 0)
        iota_loc = lax.broadcasted_iota(jnp.int32, (Sq, SKV_PER), 1)
        ki = jnp.concatenate(
            [iota_loc + SKV_PER * lax.rem(my + t, N_DEV) for t in range(N_DEV)],
            axis=1,
        )
        mask = (jnp.abs(qi - ki) <= 128) | (ki < 32) | (qi < 32)

        own_kv.wait()

        ar_rdmas = {0: [], 1: []}
        for b in range(B):
            ctx_heads = []
            for p in range(2):
                for r in kv_rdmas[(b, p)]:
                    r.wait()
                kts = [kv_all[t, b * KVROWS:b * KVROWS + SKV_PER,
                              p * HP:(p + 1) * HP] for t in range(N_DEV)]
                vts = [kv_all[t, b * KVROWS + SKV_PER:(b + 1) * KVROWS,
                              p * HP:(p + 1) * HP] for t in range(N_DEV)]
                for hh in range(2):
                    h = 2 * p + hh
                    q_bh = q[b * Sq:(b + 1) * Sq, h * DH:(h + 1) * DH]
                    k_bh = jnp.concatenate(
                        [kt[:, hh * DH:(hh + 1) * DH] for kt in kts], axis=0)
                    v_bh = jnp.concatenate(
                        [vt[:, hh * DH:(hh + 1) * DH] for vt in vts], axis=0)
                    s = lax.dot_general(
                        q_bh, k_bh, (((1,), (1,)), ((), ())),
                        preferred_element_type=jnp.float32,
                    ) * SCALE
                    s = jnp.where(mask, s, -1e9)
                    m = jnp.max(s, axis=1, keepdims=True)
                    w = jnp.exp(s - m)
                    w = (w / jnp.sum(w, axis=1,
                                     keepdims=True)).astype(jnp.bfloat16)
                    ctx_heads.append(
                        jnp.dot(w, v_bh, preferred_element_type=jnp.float32))
            ctx_b = jnp.concatenate(ctx_heads, axis=1)
            part_b = jnp.dot(ctx_b, wo_ref[...],
                             preferred_element_type=jnp.float32)
            my_part[b] = part_b.astype(jnp.bfloat16)

            for d in range(1, N_DEV):
                peer = lax.rem(my + d, N_DEV)
                slot = N_DEV - d
                r = pltpu.make_async_remote_copy(
                    src_ref=my_part.at[b],
                    dst_ref=acc.at[slot - 1, b],
                    send_sem=ar_send.at[d - 1, b],
                    recv_sem=ar_recv.at[slot - 1, b],
                    device_id=(peer,), device_id_type=pl.DeviceIdType.MESH,
                )
                r.start()
                ar_rdmas[b].append(r)

        for b in range(B):
            for r in ar_rdmas[b]:
                r.wait()
            out_ref[b] = (my_part[b].astype(jnp.float32)
                          + acc[0, b].astype(jnp.float32)
                          + acc[1, b].astype(jnp.float32)
                          + acc[2, b].astype(jnp.float32))

    return pl.pallas_call(
        body,
        out_shape=jax.ShapeDtypeStruct((B, Sq, D_MODEL), jnp.float32),
        in_specs=[pl.BlockSpec(memory_space=pltpu.VMEM)] * 5,
        out_specs=pl.BlockSpec(memory_space=pltpu.VMEM),
        scratch_shapes=[
            pltpu.VMEM((B * KVROWS, HQ * DH), jnp.bfloat16),
            pltpu.VMEM((N_DEV, B * KVROWS, HD_LOC), jnp.bfloat16),
            pltpu.VMEM((B, Sq, D_MODEL), jnp.bfloat16),
            pltpu.VMEM((N_DEV - 1, B, Sq, D_MODEL), jnp.bfloat16),
            pltpu.SemaphoreType.DMA((N_DEV - 1, B, 2)),
            pltpu.SemaphoreType.DMA((N_DEV - 1, B, 2)),
            pltpu.SemaphoreType.DMA,
            pltpu.SemaphoreType.DMA((N_DEV - 1, B)),
            pltpu.SemaphoreType.DMA((N_DEV - 1, B)),
        ],
        compiler_params=pltpu.CompilerParams(collective_id=0),
    )(x, Wq, K2, V2, Wo)
